# Initial kernel scaffold; baseline (speedup 1.0000x reference)
import jax
import jax.numpy as jnp
from jax.experimental import pallas as pl


def kernel(x, edge_index_FC, batch, x_SC, edge_index_SC, W1, b1, W2, b2, W3, b3, Wp, bp):
    raise NotImplementedError("write your pallas kernel here")


# speedup vs baseline: 2.5003x; 2.5003x over previous
import functools

import jax
import jax.numpy as jnp
from jax import lax
from jax.experimental import pallas as pl
from jax.experimental.pallas import tpu as pltpu
from jax.experimental.pallas import tpu_sc as plsc

N = 10000
E = 320000
D = 128
G = 16

NPAD = 10240
EPT = 20480
NCH = EPT // 128
ROWS_PER_TILE = NPAD // 16

@functools.cache
def _mesh():
    return plsc.VectorSubcoreMesh(core_axis_name="c", subcore_axis_name="s")


SUP = 16
NSUP = NCH // SUP


def _sc_agg_body(h_hbm, src_hbm, dst_hbm, zeros_hbm, out_hbm,
                 acc, src_v, dst_v, rows_v, sem):
    c = lax.axis_index("c")
    s = lax.axis_index("s")
    base = s * ROWS_PER_TILE
    pltpu.sync_copy(zeros_hbm, acc.at[pl.ds(base, ROWS_PER_TILE)])
    plsc.subcore_barrier()

    def outer(t, carry):
        pltpu.sync_copy(src_hbm.at[c, s, pl.ds(t * SUP, SUP)], src_v)
        pltpu.sync_copy(dst_hbm.at[c, s, pl.ds(t * SUP, SUP)], dst_v)

        def body(j, carry2):
            pltpu.async_copy(h_hbm.at[src_v.at[j]], rows_v, sem).wait()
            pltpu.sync_copy(rows_v, acc.at[dst_v.at[j]], add=True)
            return carry2

        return lax.fori_loop(0, SUP, body, carry)

    lax.fori_loop(0, NSUP, outer, 0)
    plsc.subcore_barrier()
    pltpu.sync_copy(acc.at[pl.ds(base, ROWS_PER_TILE)],
                    out_hbm.at[pl.ds(c * NPAD + base, ROWS_PER_TILE)])


@functools.partial(jax.jit, static_argnums=())
def _sc_agg(h, src, dst, zeros):
    k = pl.kernel(
        _sc_agg_body,
        mesh=_mesh(),
        out_type=jax.ShapeDtypeStruct((2 * NPAD, D), jnp.float32),
        scratch_types=[
            pltpu.VMEM_SHARED((NPAD, D), jnp.float32),
            pltpu.VMEM((SUP, 128), jnp.int32),
            pltpu.VMEM((SUP, 128), jnp.int32),
            pltpu.VMEM((128, D), jnp.float32),
            pltpu.SemaphoreType.DMA,
        ],
    )
    return k(h, src, dst, zeros)


BN = 2048


def _gin_mm_body(h_ref, agg_ref, w_ref, b_ref, o_ref):
    m = h_ref[...] + agg_ref[...]
    o_ref[...] = jnp.maximum(
        jnp.dot(m, w_ref[...], preferred_element_type=jnp.float32) + b_ref[...],
        0.0)


def _gin_mm(h, agg, W, b2d):
    M = h.shape[0]
    return pl.pallas_call(
        _gin_mm_body,
        grid=(M // BN,),
        in_specs=[
            pl.BlockSpec((BN, D), lambda i: (i, 0)),
            pl.BlockSpec((BN, D), lambda i: (i, 0)),
            pl.BlockSpec((D, D), lambda i: (0, 0)),
            pl.BlockSpec((1, D), lambda i: (0, 0)),
        ],
        out_specs=pl.BlockSpec((BN, D), lambda i: (i, 0)),
        out_shape=jax.ShapeDtypeStruct((M, D), jnp.float32),
    )(h, agg, W, b2d)


def _final_body(h_ref, agg_ref, w_ref, b_ref, batch_ref, wp_ref, bp_ref,
                o_ref, accp, cntp):
    i = pl.program_id(0)

    @pl.when(i == 0)
    def _():
        accp[...] = jnp.zeros_like(accp)
        cntp[...] = jnp.zeros_like(cntp)

    m = h_ref[...] + agg_ref[...]
    h3 = jnp.maximum(
        jnp.dot(m, w_ref[...], preferred_element_type=jnp.float32) + b_ref[...],
        0.0)
    bid = batch_ref[...].reshape(1, BN)
    oh = (lax.broadcasted_iota(jnp.int32, (G, BN), 0) == bid).astype(jnp.float32)
    accp[...] += jnp.dot(oh, h3, preferred_element_type=jnp.float32)
    cntp[...] += jnp.broadcast_to(jnp.sum(oh, axis=1, keepdims=True), (G, D))

    @pl.when(i == pl.num_programs(0) - 1)
    def _():
        mean = accp[...] / jnp.maximum(cntp[...], 1.0)
        logits = jnp.dot(mean, wp_ref[...],
                         preferred_element_type=jnp.float32) + bp_ref[...]
        o_ref[...] = jax.nn.sigmoid(logits)


def _final_tc(h, agg, W, b2d, batch3d, wp_pad, bp_pad):
    M = h.shape[0]
    return pl.pallas_call(
        _final_body,
        grid=(M // BN,),
        in_specs=[
            pl.BlockSpec((BN, D), lambda i: (i, 0)),
            pl.BlockSpec((BN, D), lambda i: (i, 0)),
            pl.BlockSpec((D, D), lambda i: (0, 0)),
            pl.BlockSpec((1, D), lambda i: (0, 0)),
            pl.BlockSpec((1, 1, BN), lambda i: (i, 0, 0)),
            pl.BlockSpec((D, D), lambda i: (0, 0)),
            pl.BlockSpec((1, D), lambda i: (0, 0)),
        ],
        out_specs=pl.BlockSpec((G, D), lambda i: (0, 0)),
        out_shape=jax.ShapeDtypeStruct((G, D), jnp.float32),
        scratch_shapes=[
            pltpu.VMEM((G, D), jnp.float32),
            pltpu.VMEM((G, D), jnp.float32),
        ],
    )(h, agg, W, b2d, batch3d, wp_pad, bp_pad)


def kernel(x, edge_index_FC, batch, x_SC, edge_index_SC,
           W1, b1, W2, b2, W3, b3, Wp, bp):
    f32 = jnp.float32
    xpad = jnp.zeros((NPAD, D), f32).at[:N].set(x)
    xscpad = jnp.zeros((NPAD, D), f32).at[:N].set(x_SC)
    H = jnp.concatenate([xpad, xscpad], axis=0)

    pad_e = 16 * EPT - E
    def prep(ei, src_off):
        src = jnp.concatenate([ei[0] + src_off,
                               jnp.full((pad_e,), src_off, jnp.int32)])
        dst = jnp.concatenate([ei[1], jnp.full((pad_e,), N, jnp.int32)])
        return src.reshape(16, NCH, 128), dst.reshape(16, NCH, 128)

    src_fc, dst_fc = prep(edge_index_FC, 0)
    src_sc, dst_sc = prep(edge_index_SC, NPAD)
    src_all = jnp.stack([src_fc, src_sc])
    dst_all = jnp.stack([dst_fc, dst_sc])

    zeros = jnp.zeros((ROWS_PER_TILE, D), f32)
    batch_pad = jnp.concatenate([batch, jnp.full((NPAD - N,), G, jnp.int32)])
    batch3d = jnp.tile(batch_pad, 2).reshape(2 * NPAD // BN, 1, BN)

    b1p = b1.reshape(1, D)
    b2p = b2.reshape(1, D)
    b3p = b3.reshape(1, D)
    wp_pad = jnp.zeros((D, D), f32).at[:, :1].set(Wp)
    bp_pad = jnp.broadcast_to(bp.reshape(1, 1), (1, D))

    AGG = _sc_agg(H, src_all, dst_all, zeros)
    H = _gin_mm(H, AGG, W1, b1p)
    AGG = _sc_agg(H, src_all, dst_all, zeros)
    H = _gin_mm(H, AGG, W2, b2p)
    AGG = _sc_agg(H, src_all, dst_all, zeros)
    out = _final_tc(H, AGG, W3, b3p, batch3d, wp_pad, bp_pad)
    return out[:, :1]


# speedup vs baseline: 2.8264x; 1.1304x over previous
import functools

import jax
import jax.numpy as jnp
from jax import lax
from jax.experimental import pallas as pl
from jax.experimental.pallas import tpu as pltpu
from jax.experimental.pallas import tpu_sc as plsc

N = 10000
E = 320000
D = 128
G = 16

NPAD = 10240
EPT = 20480
NCH = EPT // 128
ROWS_PER_TILE = NPAD // 16

@functools.cache
def _mesh():
    return plsc.VectorSubcoreMesh(core_axis_name="c", subcore_axis_name="s")


SUP = 16
NSUP = NCH // SUP


def _sc_agg_body(h_hbm, src_hbm, dst_hbm, zeros_hbm, out_hbm,
                 acc, src_v, dst_v, rows0, rows1, gsem, ssem):
    c = lax.axis_index("c")
    s = lax.axis_index("s")
    base = s * ROWS_PER_TILE
    pltpu.sync_copy(zeros_hbm, acc.at[pl.ds(base, ROWS_PER_TILE)])
    plsc.subcore_barrier()

    def wait_gather():
        pltpu.make_async_copy(h_hbm.at[pl.ds(0, 128)], rows0, gsem).wait()

    def wait_scatter():
        pltpu.make_async_copy(rows0, acc.at[pl.ds(0, 128)], ssem).wait()

    pltpu.sync_copy(src_hbm.at[c, s, pl.ds(0, SUP)], src_v)
    pltpu.sync_copy(dst_hbm.at[c, s, pl.ds(0, SUP)], dst_v)
    pltpu.async_copy(h_hbm.at[src_v.at[0]], rows0, gsem)
    wait_gather()
    pltpu.async_copy(h_hbm.at[src_v.at[1]], rows1, gsem)
    pltpu.async_copy(rows0, acc.at[dst_v.at[0]], ssem, add=True)

    def body(j, carry):
        b = j % 2
        wait_gather()
        wait_scatter()
        @pl.when(j % SUP == 0)
        def _():
            pltpu.sync_copy(
                dst_hbm.at[c, s, pl.ds(pl.multiple_of(j, SUP), SUP)], dst_v)

        @pl.when(j % SUP == SUP - 1)
        def _():
            pltpu.sync_copy(
                src_hbm.at[c, s, pl.ds(pl.multiple_of(j + 1, SUP), SUP)],
                src_v)

        @pl.when(b == 0)
        def _():
            pltpu.async_copy(h_hbm.at[src_v.at[(j + 1) % SUP]], rows1, gsem)
            pltpu.async_copy(rows0, acc.at[dst_v.at[j % SUP]], ssem, add=True)

        @pl.when(b == 1)
        def _():
            pltpu.async_copy(h_hbm.at[src_v.at[(j + 1) % SUP]], rows0, gsem)
            pltpu.async_copy(rows1, acc.at[dst_v.at[j % SUP]], ssem, add=True)

        return carry

    lax.fori_loop(1, NCH - 1, body, 0)
    wait_gather()
    wait_scatter()
    pltpu.async_copy(rows1, acc.at[dst_v.at[SUP - 1]], ssem, add=True)
    wait_scatter()
    plsc.subcore_barrier()
    pltpu.sync_copy(acc.at[pl.ds(base, ROWS_PER_TILE)],
                    out_hbm.at[pl.ds(c * NPAD + base, ROWS_PER_TILE)])


@functools.partial(jax.jit, static_argnums=())
def _sc_agg(h, src, dst, zeros):
    k = pl.kernel(
        _sc_agg_body,
        mesh=_mesh(),
        out_type=jax.ShapeDtypeStruct((2 * NPAD, D), jnp.float32),
        scratch_types=[
            pltpu.VMEM_SHARED((NPAD, D), jnp.float32),
            pltpu.VMEM((SUP, 128), jnp.int32),
            pltpu.VMEM((SUP, 128), jnp.int32),
            pltpu.VMEM((128, D), jnp.float32),
            pltpu.VMEM((128, D), jnp.float32),
            pltpu.SemaphoreType.DMA,
            pltpu.SemaphoreType.DMA,
        ],
    )
    return k(h, src, dst, zeros)


BN = 2048


def _gin_mm_body(h_ref, agg_ref, w_ref, b_ref, o_ref):
    m = h_ref[...] + agg_ref[...]
    o_ref[...] = jnp.maximum(
        jnp.dot(m, w_ref[...], preferred_element_type=jnp.float32) + b_ref[...],
        0.0)


def _gin_mm(h, agg, W, b2d):
    M = h.shape[0]
    return pl.pallas_call(
        _gin_mm_body,
        grid=(M // BN,),
        in_specs=[
            pl.BlockSpec((BN, D), lambda i: (i, 0)),
            pl.BlockSpec((BN, D), lambda i: (i, 0)),
            pl.BlockSpec((D, D), lambda i: (0, 0)),
            pl.BlockSpec((1, D), lambda i: (0, 0)),
        ],
        out_specs=pl.BlockSpec((BN, D), lambda i: (i, 0)),
        out_shape=jax.ShapeDtypeStruct((M, D), jnp.float32),
    )(h, agg, W, b2d)


def _final_body(h_ref, agg_ref, w_ref, b_ref, batch_ref, wp_ref, bp_ref,
                o_ref, accp, cntp):
    i = pl.program_id(0)

    @pl.when(i == 0)
    def _():
        accp[...] = jnp.zeros_like(accp)
        cntp[...] = jnp.zeros_like(cntp)

    m = h_ref[...] + agg_ref[...]
    h3 = jnp.maximum(
        jnp.dot(m, w_ref[...], preferred_element_type=jnp.float32) + b_ref[...],
        0.0)
    bid = batch_ref[...].reshape(1, BN)
    oh = (lax.broadcasted_iota(jnp.int32, (G, BN), 0) == bid).astype(jnp.float32)
    accp[...] += jnp.dot(oh, h3, preferred_element_type=jnp.float32)
    cntp[...] += jnp.broadcast_to(jnp.sum(oh, axis=1, keepdims=True), (G, D))

    @pl.when(i == pl.num_programs(0) - 1)
    def _():
        mean = accp[...] / jnp.maximum(cntp[...], 1.0)
        logits = jnp.dot(mean, wp_ref[...],
                         preferred_element_type=jnp.float32) + bp_ref[...]
        o_ref[...] = jax.nn.sigmoid(logits)


def _final_tc(h, agg, W, b2d, batch3d, wp_pad, bp_pad):
    M = h.shape[0]
    return pl.pallas_call(
        _final_body,
        grid=(M // BN,),
        in_specs=[
            pl.BlockSpec((BN, D), lambda i: (i, 0)),
            pl.BlockSpec((BN, D), lambda i: (i, 0)),
            pl.BlockSpec((D, D), lambda i: (0, 0)),
            pl.BlockSpec((1, D), lambda i: (0, 0)),
            pl.BlockSpec((1, 1, BN), lambda i: (i, 0, 0)),
            pl.BlockSpec((D, D), lambda i: (0, 0)),
            pl.BlockSpec((1, D), lambda i: (0, 0)),
        ],
        out_specs=pl.BlockSpec((G, D), lambda i: (0, 0)),
        out_shape=jax.ShapeDtypeStruct((G, D), jnp.float32),
        scratch_shapes=[
            pltpu.VMEM((G, D), jnp.float32),
            pltpu.VMEM((G, D), jnp.float32),
        ],
    )(h, agg, W, b2d, batch3d, wp_pad, bp_pad)


def kernel(x, edge_index_FC, batch, x_SC, edge_index_SC,
           W1, b1, W2, b2, W3, b3, Wp, bp):
    f32 = jnp.float32
    xpad = jnp.zeros((NPAD, D), f32).at[:N].set(x)
    xscpad = jnp.zeros((NPAD, D), f32).at[:N].set(x_SC)
    H = jnp.concatenate([xpad, xscpad], axis=0)

    pad_e = 16 * EPT - E
    def prep(ei, src_off):
        src = jnp.concatenate([ei[0] + src_off,
                               jnp.full((pad_e,), src_off, jnp.int32)])
        dst = jnp.concatenate([ei[1], jnp.full((pad_e,), N, jnp.int32)])
        return src.reshape(16, NCH, 128), dst.reshape(16, NCH, 128)

    src_fc, dst_fc = prep(edge_index_FC, 0)
    src_sc, dst_sc = prep(edge_index_SC, NPAD)
    src_all = jnp.stack([src_fc, src_sc])
    dst_all = jnp.stack([dst_fc, dst_sc])

    zeros = jnp.zeros((ROWS_PER_TILE, D), f32)
    batch_pad = jnp.concatenate([batch, jnp.full((NPAD - N,), G, jnp.int32)])
    batch3d = jnp.tile(batch_pad, 2).reshape(2 * NPAD // BN, 1, BN)

    b1p = b1.reshape(1, D)
    b2p = b2.reshape(1, D)
    b3p = b3.reshape(1, D)
    wp_pad = jnp.zeros((D, D), f32).at[:, :1].set(Wp)
    bp_pad = jnp.broadcast_to(bp.reshape(1, 1), (1, D))

    AGG = _sc_agg(H, src_all, dst_all, zeros)
    H = _gin_mm(H, AGG, W1, b1p)
    AGG = _sc_agg(H, src_all, dst_all, zeros)
    H = _gin_mm(H, AGG, W2, b2p)
    AGG = _sc_agg(H, src_all, dst_all, zeros)
    out = _final_tc(H, AGG, W3, b3p, batch3d, wp_pad, bp_pad)
    return out[:, :1]


# speedup vs baseline: 4.0576x; 1.4356x over previous
import functools

import jax
import jax.numpy as jnp
from jax import lax
from jax.experimental import pallas as pl
from jax.experimental.pallas import tpu as pltpu
from jax.experimental.pallas import tpu_sc as plsc

N = 10000
E = 320000
D = 128
G = 16

NPAD = 10240
EPT = 20480
NCH = EPT // 128
ROWS_PER_TILE = NPAD // 16
SUP = 16
NSUP = NCH // SUP


@functools.cache
def _mesh():
    return plsc.VectorSubcoreMesh(core_axis_name="c", subcore_axis_name="s")


def _sc_agg_body(h_hbm, src_hbm, dst_hbm, zeros_hbm, out_hbm,
                 acc, src_v, dst_v, rows0, rows1, gsem, ssem):
    c = lax.axis_index("c")
    s = lax.axis_index("s")
    base = s * ROWS_PER_TILE
    pltpu.sync_copy(zeros_hbm, acc.at[pl.ds(base, ROWS_PER_TILE)])
    plsc.subcore_barrier()

    def wait_gather():
        pltpu.make_async_copy(h_hbm.at[pl.ds(0, 128)], rows0, gsem).wait()

    def wait_scatter():
        pltpu.make_async_copy(rows0, acc.at[pl.ds(0, 128)], ssem).wait()

    pltpu.sync_copy(src_hbm.at[c, s, pl.ds(0, SUP)], src_v)
    pltpu.sync_copy(dst_hbm.at[c, s, pl.ds(0, SUP)], dst_v)
    pltpu.async_copy(h_hbm.at[src_v.at[0]], rows0, gsem)
    wait_gather()
    pltpu.async_copy(h_hbm.at[src_v.at[1]], rows1, gsem)
    pltpu.async_copy(rows0, acc.at[dst_v.at[0]], ssem, add=True)

    def body(j, carry):
        b = j % 2
        wait_gather()
        wait_scatter()
        @pl.when(j % SUP == 0)
        def _():
            pltpu.sync_copy(
                dst_hbm.at[c, s, pl.ds(pl.multiple_of(j, SUP), SUP)], dst_v)

        @pl.when(j % SUP == SUP - 1)
        def _():
            pltpu.sync_copy(
                src_hbm.at[c, s, pl.ds(pl.multiple_of(j + 1, SUP), SUP)],
                src_v)

        @pl.when(b == 0)
        def _():
            pltpu.async_copy(h_hbm.at[src_v.at[(j + 1) % SUP]], rows1, gsem)
            pltpu.async_copy(rows0, acc.at[dst_v.at[j % SUP]], ssem, add=True)

        @pl.when(b == 1)
        def _():
            pltpu.async_copy(h_hbm.at[src_v.at[(j + 1) % SUP]], rows0, gsem)
            pltpu.async_copy(rows1, acc.at[dst_v.at[j % SUP]], ssem, add=True)

        return carry

    lax.fori_loop(1, NCH - 1, body, 0)
    wait_gather()
    wait_scatter()
    pltpu.async_copy(rows1, acc.at[dst_v.at[SUP - 1]], ssem, add=True)
    wait_scatter()
    plsc.subcore_barrier()
    pltpu.sync_copy(acc.at[pl.ds(base, ROWS_PER_TILE)],
                    out_hbm.at[pl.ds(c * NPAD + base, ROWS_PER_TILE)])


def _sc_agg(h16, src, dst, zeros16):
    k = pl.kernel(
        _sc_agg_body,
        mesh=_mesh(),
        out_type=jax.ShapeDtypeStruct((2 * NPAD, D), jnp.bfloat16),
        compiler_params=pltpu.CompilerParams(use_tc_tiling_on_sc=False),
        scratch_types=[
            pltpu.VMEM_SHARED((NPAD, D), jnp.bfloat16),
            pltpu.VMEM((SUP, 128), jnp.int32),
            pltpu.VMEM((SUP, 128), jnp.int32),
            pltpu.VMEM((128, D), jnp.bfloat16),
            pltpu.VMEM((128, D), jnp.bfloat16),
            pltpu.SemaphoreType.DMA,
            pltpu.SemaphoreType.DMA,
        ],
    )
    return k(h16, src, dst, zeros16)


BN = 2048


def _gin_mm_body(h_ref, agg_ref, w_ref, b_ref, o_ref, o16_ref):
    m = h_ref[...] + agg_ref[...].astype(jnp.float32)
    o = jnp.maximum(
        jnp.dot(m, w_ref[...], preferred_element_type=jnp.float32) + b_ref[...],
        0.0)
    o_ref[...] = o
    o16_ref[...] = o.astype(jnp.bfloat16)


def _gin_mm(h, agg16, W, b2d):
    M = h.shape[0]
    return pl.pallas_call(
        _gin_mm_body,
        grid=(M // BN,),
        in_specs=[
            pl.BlockSpec((BN, D), lambda i: (i, 0)),
            pl.BlockSpec((BN, D), lambda i: (i, 0)),
            pl.BlockSpec((D, D), lambda i: (0, 0)),
            pl.BlockSpec((1, D), lambda i: (0, 0)),
        ],
        out_specs=[pl.BlockSpec((BN, D), lambda i: (i, 0)),
                   pl.BlockSpec((BN, D), lambda i: (i, 0))],
        out_shape=[jax.ShapeDtypeStruct((M, D), jnp.float32),
                   jax.ShapeDtypeStruct((M, D), jnp.bfloat16)],
    )(h, agg16, W, b2d)


def _final_body(h_ref, agg_ref, w_ref, b_ref, batch_ref, wp_ref, bp_ref,
                o_ref, accp, cntp):
    i = pl.program_id(0)

    @pl.when(i == 0)
    def _():
        accp[...] = jnp.zeros_like(accp)
        cntp[...] = jnp.zeros_like(cntp)

    m = h_ref[...] + agg_ref[...].astype(jnp.float32)
    h3 = jnp.maximum(
        jnp.dot(m, w_ref[...], preferred_element_type=jnp.float32) + b_ref[...],
        0.0)
    bid = batch_ref[...].reshape(1, BN)
    oh = (lax.broadcasted_iota(jnp.int32, (G, BN), 0) == bid).astype(jnp.float32)
    accp[...] += jnp.dot(oh, h3, preferred_element_type=jnp.float32)
    cntp[...] += jnp.broadcast_to(jnp.sum(oh, axis=1, keepdims=True), (G, D))

    @pl.when(i == pl.num_programs(0) - 1)
    def _():
        mean = accp[...] / jnp.maximum(cntp[...], 1.0)
        logits = jnp.dot(mean, wp_ref[...],
                         preferred_element_type=jnp.float32) + bp_ref[...]
        o_ref[...] = jax.nn.sigmoid(logits)


def _final_tc(h, agg16, W, b2d, batch3d, wp_pad, bp_pad):
    M = h.shape[0]
    return pl.pallas_call(
        _final_body,
        grid=(M // BN,),
        in_specs=[
            pl.BlockSpec((BN, D), lambda i: (i, 0)),
            pl.BlockSpec((BN, D), lambda i: (i, 0)),
            pl.BlockSpec((D, D), lambda i: (0, 0)),
            pl.BlockSpec((1, D), lambda i: (0, 0)),
            pl.BlockSpec((1, 1, BN), lambda i: (i, 0, 0)),
            pl.BlockSpec((D, D), lambda i: (0, 0)),
            pl.BlockSpec((1, D), lambda i: (0, 0)),
        ],
        out_specs=pl.BlockSpec((G, D), lambda i: (0, 0)),
        out_shape=jax.ShapeDtypeStruct((G, D), jnp.float32),
        scratch_shapes=[
            pltpu.VMEM((G, D), jnp.float32),
            pltpu.VMEM((G, D), jnp.float32),
        ],
    )(h, agg16, W, b2d, batch3d, wp_pad, bp_pad)


def kernel(x, edge_index_FC, batch, x_SC, edge_index_SC,
           W1, b1, W2, b2, W3, b3, Wp, bp):
    f32 = jnp.float32
    xpad = jnp.zeros((NPAD, D), f32).at[:N].set(x)
    xscpad = jnp.zeros((NPAD, D), f32).at[:N].set(x_SC)
    H = jnp.concatenate([xpad, xscpad], axis=0)

    pad_e = 16 * EPT - E
    def prep(ei, src_off):
        src = jnp.concatenate([ei[0] + src_off,
                               jnp.full((pad_e,), src_off, jnp.int32)])
        dst = jnp.concatenate([ei[1], jnp.full((pad_e,), N, jnp.int32)])
        return src.reshape(16, NCH, 128), dst.reshape(16, NCH, 128)

    src_fc, dst_fc = prep(edge_index_FC, 0)
    src_sc, dst_sc = prep(edge_index_SC, NPAD)
    src_all = jnp.stack([src_fc, src_sc])
    dst_all = jnp.stack([dst_fc, dst_sc])

    zeros16 = jnp.zeros((ROWS_PER_TILE, D), jnp.bfloat16)
    batch_pad = jnp.concatenate([batch, jnp.full((NPAD - N,), G, jnp.int32)])
    batch3d = jnp.tile(batch_pad, 2).reshape(2 * NPAD // BN, 1, BN)

    b1p = b1.reshape(1, D)
    b2p = b2.reshape(1, D)
    b3p = b3.reshape(1, D)
    wp_pad = jnp.zeros((D, D), f32).at[:, :1].set(Wp)
    bp_pad = jnp.broadcast_to(bp.reshape(1, 1), (1, D))

    AGG = _sc_agg(H.astype(jnp.bfloat16), src_all, dst_all, zeros16)
    H, H16 = _gin_mm(H, AGG, W1, b1p)
    AGG = _sc_agg(H16, src_all, dst_all, zeros16)
    H, H16 = _gin_mm(H, AGG, W2, b2p)
    AGG = _sc_agg(H16, src_all, dst_all, zeros16)
    out = _final_tc(H, AGG, W3, b3p, batch3d, wp_pad, bp_pad)
    return out[:, :1]
